# baseline (device time: 36907 ns/iter reference)
import os

import jax
import jax.numpy as jnp
from jax import lax
from jax.experimental import pallas as pl
from jax.experimental.pallas import tpu as pltpu

_DEBUG_NO_RDMA = os.environ.get("SCBAND_DEBUG_NO_RDMA") == "1"

N_DEV = 4
B = 2
S = 512
W = 128
HQ = 8
DH = 64
HD = HQ * DH
E = 768
S_EXT = S + 2 * W
S_GLOBAL = N_DEV * S
QB = S // W
KSLAB = 3 * W


def kernel(x, Wq, K_ext, V_ext, Wo):
    K2 = K_ext.reshape(B, S, HD)
    V2 = V_ext.reshape(B, S, HD)

    def body(x_ref, wq_ref, k_ref, v_ref, wo_ref, out_ref,
             keff, veff, send_sems, recv_sems):
        my = lax.axis_index("i")
        left = lax.rem(my + N_DEV - 1, N_DEV)
        right = lax.rem(my + 1, N_DEV)

        keff[:, W:W + S, :] = k_ref[...].astype(jnp.bfloat16)
        veff[:, W:W + S, :] = v_ref[...].astype(jnp.bfloat16)

        barrier = pltpu.get_barrier_semaphore()
        for nbr in (left, right):
            pl.semaphore_signal(
                barrier, inc=1,
                device_id=(nbr,), device_id_type=pl.DeviceIdType.MESH,
            )
        pl.semaphore_wait(barrier, 2)

        flows = [] if _DEBUG_NO_RDMA else [
            (buf, b, src_lo, dst_lo, tgt)
            for b in range(B)
            for (buf, src_lo, dst_lo, tgt) in [
                (keff, S, 0, right),
                (veff, S, 0, right),
                (keff, W, S + W, left),
                (veff, W, S + W, left),
            ]
        ]
        rdmas = []
        for idx, (buf, b, src_lo, dst_lo, tgt) in enumerate(flows):
            rdma = pltpu.make_async_remote_copy(
                src_ref=buf.at[b, pl.ds(src_lo, W), :],
                dst_ref=buf.at[b, pl.ds(dst_lo, W), :],
                send_sem=send_sems.at[idx],
                recv_sem=recv_sems.at[idx],
                device_id=(tgt,),
                device_id_type=pl.DeviceIdType.MESH,
            )
            rdma.start()
            rdmas.append(rdma)
        left_halo = None if _DEBUG_NO_RDMA else [
            (rdmas[b * 4 + 0], rdmas[b * 4 + 1]) for b in range(B)]
        right_halo = None if _DEBUG_NO_RDMA else [
            (rdmas[b * 4 + 2], rdmas[b * 4 + 3]) for b in range(B)]

        wq_b = wq_ref[...].astype(jnp.bfloat16)
        wo_b = wo_ref[...].astype(jnp.bfloat16)
        q_all = []
        for b in range(B):
            xb = x_ref[b].astype(jnp.bfloat16)
            qb = lax.dot_general(
                xb, wq_b, (((1,), (0,)), ((), ())),
                preferred_element_type=jnp.float32,
            )
            q_all.append(qb.astype(jnp.bfloat16))

        r_io = lax.broadcasted_iota(jnp.int32, (W, KSLAB), 0)
        c_io = lax.broadcasted_iota(jnp.int32, (W, KSLAB), 1)
        band = (c_io - r_io >= 0) & (c_io - r_io <= 2 * W)
        biases = []
        for qb in range(QB):
            kpos = my * S - W + qb * W + c_io
            valid = band & (kpos >= 0) & (kpos < S_GLOBAL)
            biases.append(jnp.where(valid, 0.0, -1e9).astype(jnp.float32))

        def block(b, qb):
            j0 = qb * W
            kslab = keff[b, pl.ds(j0, KSLAB), :]
            vslab = veff[b, pl.ds(j0, KSLAB), :]
            ctx_heads = []
            for h in range(HQ):
                qh = q_all[b][qb * W:(qb + 1) * W, h * DH:(h + 1) * DH]
                kh = kslab[:, h * DH:(h + 1) * DH]
                vh = vslab[:, h * DH:(h + 1) * DH]
                s = lax.dot_general(
                    qh, kh, (((1,), (1,)), ((), ())),
                    preferred_element_type=jnp.float32,
                )
                s = s * 0.125 + biases[qb]
                m = jnp.max(s, axis=-1, keepdims=True)
                e = jnp.exp(s - m)
                w = e / jnp.sum(e, axis=-1, keepdims=True)
                ctx_heads.append(lax.dot_general(
                    w.astype(jnp.bfloat16), vh, (((1,), (0,)), ((), ())),
                    preferred_element_type=jnp.float32,
                ))
            ctx = jnp.concatenate(ctx_heads, axis=1).astype(jnp.bfloat16)
            out_ref[b, pl.ds(qb * W, W), :] = lax.dot_general(
                ctx, wo_b, (((1,), (0,)), ((), ())),
                preferred_element_type=jnp.float32,
            )

        for b in range(B):
            block(b, 1)
            block(b, 2)
        for b in range(B):
            if left_halo is not None:
                for r in left_halo[b]:
                    r.wait_recv()
            block(b, 0)
        for b in range(B):
            if right_halo is not None:
                for r in right_halo[b]:
                    r.wait_recv()
            block(b, QB - 1)

        for r in rdmas:
            r.wait_send()

    return pl.pallas_call(
        body,
        out_shape=jax.ShapeDtypeStruct((B, S, E), jnp.float32),
        in_specs=[pl.BlockSpec(memory_space=pltpu.VMEM)] * 5,
        out_specs=pl.BlockSpec(memory_space=pltpu.VMEM),
        scratch_shapes=[
            pltpu.VMEM((B, S_EXT, HD), jnp.bfloat16),
            pltpu.VMEM((B, S_EXT, HD), jnp.bfloat16),
            pltpu.SemaphoreType.DMA((4 * B,)),
            pltpu.SemaphoreType.DMA((4 * B,)),
        ],
        compiler_params=pltpu.CompilerParams(collective_id=0),
    )(x, Wq, K2, V2, Wo)


# device time: 26128 ns/iter; 1.4125x vs baseline; 1.4125x over previous
import os

import jax
import jax.numpy as jnp
from jax import lax
from jax.experimental import pallas as pl
from jax.experimental.pallas import tpu as pltpu

_DEBUG_NO_RDMA = os.environ.get("SCBAND_DEBUG_NO_RDMA") == "1"
_DEBUG_SKIP = os.environ.get("SCBAND_DEBUG_SKIP", "")

N_DEV = 4
B = 2
S = 512
W = 128
HQ = 8
DH = 64
HD = HQ * DH
E = 768
S_EXT = S + 2 * W
S_GLOBAL = N_DEV * S
QBLK = 256
KSLAB = QBLK + 2 * W
NQB = S // QBLK


def kernel(x, Wq, K_ext, V_ext, Wo):
    K2 = K_ext.reshape(B, S, HD)
    V2 = V_ext.reshape(B, S, HD)

    def body(x_ref, wq_ref, k_ref, v_ref, wo_ref, out_ref,
             keff, veff, send_sems, recv_sems):
        my = lax.axis_index("i")
        left = lax.rem(my + N_DEV - 1, N_DEV)
        right = lax.rem(my + 1, N_DEV)

        keff[:, W:W + S, :] = k_ref[...].astype(jnp.bfloat16)
        veff[:, W:W + S, :] = v_ref[...].astype(jnp.bfloat16)

        barrier = pltpu.get_barrier_semaphore()
        for nbr in (left, right):
            pl.semaphore_signal(
                barrier, inc=1,
                device_id=(nbr,), device_id_type=pl.DeviceIdType.MESH,
            )
        pl.semaphore_wait(barrier, 2)

        flows = [] if _DEBUG_NO_RDMA else [
            (buf, b, src_lo, dst_lo, tgt)
            for b in range(B)
            for (buf, src_lo, dst_lo, tgt) in [
                (keff, S, 0, right),
                (veff, S, 0, right),
                (keff, W, S + W, left),
                (veff, W, S + W, left),
            ]
        ]
        rdmas = []
        for idx, (buf, b, src_lo, dst_lo, tgt) in enumerate(flows):
            rdma = pltpu.make_async_remote_copy(
                src_ref=buf.at[b, pl.ds(src_lo, W), :],
                dst_ref=buf.at[b, pl.ds(dst_lo, W), :],
                send_sem=send_sems.at[idx],
                recv_sem=recv_sems.at[idx],
                device_id=(tgt,),
                device_id_type=pl.DeviceIdType.MESH,
            )
            rdma.start()
            rdmas.append(rdma)
        left_halo = None if _DEBUG_NO_RDMA else [
            (rdmas[b * 4 + 0], rdmas[b * 4 + 1]) for b in range(B)]
        right_halo = None if _DEBUG_NO_RDMA else [
            (rdmas[b * 4 + 2], rdmas[b * 4 + 3]) for b in range(B)]

        wq_b = wq_ref[...].astype(jnp.bfloat16)
        wo_b = wo_ref[...].astype(jnp.bfloat16)
        q_all = []
        for b in range(B):
            xb = x_ref[b].astype(jnp.bfloat16)
            qb = lax.dot_general(
                xb, wq_b, (((1,), (0,)), ((), ())),
                preferred_element_type=jnp.float32,
            )
            q_all.append(qb.astype(jnp.bfloat16))

        r_io = lax.broadcasted_iota(jnp.int32, (QBLK, KSLAB), 0)
        c_io = lax.broadcasted_iota(jnp.int32, (QBLK, KSLAB), 1)
        band = (c_io - r_io >= 0) & (c_io - r_io <= 2 * W)
        biases = []
        for qb in range(NQB):
            kpos = my * S - W + qb * QBLK + c_io
            valid = band & (kpos >= 0) & (kpos < S_GLOBAL)
            biases.append(jnp.where(valid, 0.0, -1e9).astype(jnp.float32))

        def block(b, qb):
            j0 = qb * QBLK
            kslab = keff[b, pl.ds(j0, KSLAB), :]
            vslab = veff[b, pl.ds(j0, KSLAB), :]
            if _DEBUG_SKIP == "attn":
                ctx = q_all[b][qb * QBLK:(qb + 1) * QBLK, :]
                out_ref[b, pl.ds(qb * QBLK, QBLK), :] = lax.dot_general(
                    ctx, wo_b, (((1,), (0,)), ((), ())),
                    preferred_element_type=jnp.float32,
                )
                return
            ctx_heads = []
            for h in range(HQ):
                qh = q_all[b][qb * QBLK:(qb + 1) * QBLK, h * DH:(h + 1) * DH]
                kh = kslab[:, h * DH:(h + 1) * DH]
                vh = vslab[:, h * DH:(h + 1) * DH]
                s = lax.dot_general(
                    qh, kh, (((1,), (1,)), ((), ())),
                    preferred_element_type=jnp.float32,
                )
                if _DEBUG_SKIP == "softmax":
                    e = s.astype(jnp.bfloat16)
                    inv = 1.0
                else:
                    e = jnp.exp(s * 0.125 + biases[qb])
                    inv = 1.0 / lax.dot_general(
                        e.astype(jnp.bfloat16),
                        jnp.ones((KSLAB, 1), jnp.bfloat16),
                        (((1,), (0,)), ((), ())),
                        preferred_element_type=jnp.float32,
                    )
                    e = e.astype(jnp.bfloat16)
                ctx_u = lax.dot_general(
                    e, vh, (((1,), (0,)), ((), ())),
                    preferred_element_type=jnp.float32,
                )
                ctx_heads.append((ctx_u * inv).astype(jnp.bfloat16))
            ctx = jnp.concatenate(ctx_heads, axis=1)
            out_ref[b, pl.ds(qb * QBLK, QBLK), :] = lax.dot_general(
                ctx, wo_b, (((1,), (0,)), ((), ())),
                preferred_element_type=jnp.float32,
            )

        for b in range(B):
            if left_halo is not None:
                for r in left_halo[b]:
                    r.wait_recv()
            block(b, 0)
        for b in range(B):
            if right_halo is not None:
                for r in right_halo[b]:
                    r.wait_recv()
            block(b, 1)

        for r in rdmas:
            r.wait_send()

    return pl.pallas_call(
        body,
        out_shape=jax.ShapeDtypeStruct((B, S, E), jnp.float32),
        in_specs=[pl.BlockSpec(memory_space=pltpu.VMEM)] * 5,
        out_specs=pl.BlockSpec(memory_space=pltpu.VMEM),
        scratch_shapes=[
            pltpu.VMEM((B, S_EXT, HD), jnp.bfloat16),
            pltpu.VMEM((B, S_EXT, HD), jnp.bfloat16),
            pltpu.SemaphoreType.DMA((4 * B,)),
            pltpu.SemaphoreType.DMA((4 * B,)),
        ],
        compiler_params=pltpu.CompilerParams(collective_id=0),
    )(x, Wq, K2, V2, Wo)


# device time: 24244 ns/iter; 1.5223x vs baseline; 1.0777x over previous
import os

import jax
import jax.numpy as jnp
from jax import lax
from jax.experimental import pallas as pl
from jax.experimental.pallas import tpu as pltpu

_DEBUG_NO_RDMA = os.environ.get("SCBAND_DEBUG_NO_RDMA") == "1"
_DEBUG_SKIP = os.environ.get("SCBAND_DEBUG_SKIP", "")

N_DEV = 4
B = 2
S = 512
W = 128
HQ = 8
DH = 64
HD = HQ * DH
E = 768
S_EXT = S + 2 * W
S_GLOBAL = N_DEV * S
QBLK = 256
KSLAB = QBLK + 2 * W
NQB = S // QBLK


def kernel(x, Wq, K_ext, V_ext, Wo):
    K2 = K_ext.reshape(B, S, HD)
    V2 = V_ext.reshape(B, S, HD)

    def body(x_ref, wq_ref, k_ref, v_ref, wo_ref, out_ref,
             keff, veff, send_sems, recv_sems):
        my = lax.axis_index("i")
        left = lax.rem(my + N_DEV - 1, N_DEV)
        right = lax.rem(my + 1, N_DEV)

        keff[:, W:W + S, :] = k_ref[...].astype(jnp.bfloat16)
        veff[:, W:W + S, :] = v_ref[...].astype(jnp.bfloat16)

        barrier = pltpu.get_barrier_semaphore()
        for nbr in (left, right):
            pl.semaphore_signal(
                barrier, inc=1,
                device_id=(nbr,), device_id_type=pl.DeviceIdType.MESH,
            )
        pl.semaphore_wait(barrier, 2)

        flows = [] if _DEBUG_NO_RDMA else [
            (buf, b, src_lo, dst_lo, tgt)
            for b in range(B)
            for (buf, src_lo, dst_lo, tgt) in [
                (keff, S, 0, right),
                (veff, S, 0, right),
                (keff, W, S + W, left),
                (veff, W, S + W, left),
            ]
        ]
        rdmas = []
        for idx, (buf, b, src_lo, dst_lo, tgt) in enumerate(flows):
            rdma = pltpu.make_async_remote_copy(
                src_ref=buf.at[b, pl.ds(src_lo, W), :],
                dst_ref=buf.at[b, pl.ds(dst_lo, W), :],
                send_sem=send_sems.at[idx],
                recv_sem=recv_sems.at[idx],
                device_id=(tgt,),
                device_id_type=pl.DeviceIdType.MESH,
            )
            rdma.start()
            rdmas.append(rdma)
        left_halo = None if _DEBUG_NO_RDMA else [
            (rdmas[b * 4 + 0], rdmas[b * 4 + 1]) for b in range(B)]
        right_halo = None if _DEBUG_NO_RDMA else [
            (rdmas[b * 4 + 2], rdmas[b * 4 + 3]) for b in range(B)]

        wq_b = wq_ref[...].astype(jnp.bfloat16)
        wo_b = wo_ref[...].astype(jnp.bfloat16)
        q_all = []
        for b in range(B):
            xb = x_ref[b].astype(jnp.bfloat16)
            qb = lax.dot_general(
                xb, wq_b, (((1,), (0,)), ((), ())),
                preferred_element_type=jnp.float32,
            )
            q_all.append(qb.astype(jnp.bfloat16))

        BLOCKS = {
            "left": (0, W, 3 * W),
            "mid": (W, S - 2 * W, S),
            "right": (S - W, W, 3 * W),
        }
        biases = {}
        for name, (r0, rw, kw) in BLOCKS.items():
            r_io = lax.broadcasted_iota(jnp.int32, (rw, kw), 0)
            c_io = lax.broadcasted_iota(jnp.int32, (rw, kw), 1)
            band = (c_io - r_io >= 0) & (c_io - r_io <= 2 * W)
            kpos = my * S - W + r0 + c_io
            valid = band & (kpos >= 0) & (kpos < S_GLOBAL)
            biases[name] = jnp.where(valid, 0.0, -1e9).astype(jnp.float32)

        def block(b, name):
            r0, rw, kw = BLOCKS[name]
            kslab = keff[b, pl.ds(r0, kw), :]
            vslab = veff[b, pl.ds(r0, kw), :]
            if _DEBUG_SKIP == "attn":
                ctx = q_all[b][r0:r0 + rw, :]
                out_ref[b, pl.ds(r0, rw), :] = lax.dot_general(
                    ctx, wo_b, (((1,), (0,)), ((), ())),
                    preferred_element_type=jnp.float32,
                ).astype(jnp.bfloat16)
                return
            ctx_heads = []
            for h in range(HQ):
                qh = q_all[b][r0:r0 + rw, h * DH:(h + 1) * DH]
                kh = kslab[:, h * DH:(h + 1) * DH]
                vh = vslab[:, h * DH:(h + 1) * DH]
                s = lax.dot_general(
                    qh, kh, (((1,), (1,)), ((), ())),
                    preferred_element_type=jnp.float32,
                )
                if _DEBUG_SKIP == "softmax":
                    e = s.astype(jnp.bfloat16)
                    inv = 1.0
                else:
                    e = jnp.exp(s * 0.125 + biases[name])
                    inv = 1.0 / lax.dot_general(
                        e.astype(jnp.bfloat16),
                        jnp.ones((kw, 1), jnp.bfloat16),
                        (((1,), (0,)), ((), ())),
                        preferred_element_type=jnp.float32,
                    )
                    e = e.astype(jnp.bfloat16)
                ctx_u = lax.dot_general(
                    e, vh, (((1,), (0,)), ((), ())),
                    preferred_element_type=jnp.float32,
                )
                ctx_heads.append((ctx_u * inv).astype(jnp.bfloat16))
            ctx = jnp.concatenate(ctx_heads, axis=1)
            out_ref[b, pl.ds(r0, rw), :] = lax.dot_general(
                ctx, wo_b, (((1,), (0,)), ((), ())),
                preferred_element_type=jnp.float32,
            ).astype(jnp.bfloat16)

        for b in range(B):
            block(b, "mid")
        for b in range(B):
            if left_halo is not None:
                for r in left_halo[b]:
                    r.wait_recv()
            block(b, "left")
        for b in range(B):
            if right_halo is not None:
                for r in right_halo[b]:
                    r.wait_recv()
            block(b, "right")

        for r in rdmas:
            r.wait_send()

    return pl.pallas_call(
        body,
        out_shape=jax.ShapeDtypeStruct((B, S, E), jnp.bfloat16),
        in_specs=[pl.BlockSpec(memory_space=pltpu.VMEM)] * 5,
        out_specs=pl.BlockSpec(memory_space=pltpu.VMEM),
        scratch_shapes=[
            pltpu.VMEM((B, S_EXT, HD), jnp.bfloat16),
            pltpu.VMEM((B, S_EXT, HD), jnp.bfloat16),
            pltpu.SemaphoreType.DMA((4 * B,)),
            pltpu.SemaphoreType.DMA((4 * B,)),
        ],
        compiler_params=pltpu.CompilerParams(collective_id=0),
    )(x, Wq, K2, V2, Wo)


# device time: 24039 ns/iter; 1.5353x vs baseline; 1.0085x over previous
import os

import jax
import jax.numpy as jnp
from jax import lax
from jax.experimental import pallas as pl
from jax.experimental.pallas import tpu as pltpu

_DEBUG_NO_RDMA = os.environ.get("SCBAND_DEBUG_NO_RDMA") == "1"
_DEBUG_SKIP = os.environ.get("SCBAND_DEBUG_SKIP", "")

N_DEV = 4
B = 2
S = 512
W = 128
HQ = 8
DH = 64
HD = HQ * DH
E = 768
S_EXT = S + 2 * W
S_GLOBAL = N_DEV * S
QBLK = 256
KSLAB = QBLK + 2 * W
NQB = S // QBLK


def kernel(x, Wq, K_ext, V_ext, Wo):
    K2 = K_ext.reshape(B, S, HD)
    V2 = V_ext.reshape(B, S, HD)

    def body(x_hbm, wq_hbm, k_hbm, v_hbm, wo_hbm, out_ref,
             x_ref, wq_ref, k_ref, v_ref, wo_ref,
             keff, veff, load_sems, send_sems, recv_sems):
        my = lax.axis_index("i")
        left = lax.rem(my + N_DEV - 1, N_DEV)
        right = lax.rem(my + 1, N_DEV)

        barrier = pltpu.get_barrier_semaphore()
        for nbr in (left, right):
            pl.semaphore_signal(
                barrier, inc=1,
                device_id=(nbr,), device_id_type=pl.DeviceIdType.MESH,
            )

        loads = []
        for i, (src, dst) in enumerate([
            (k_hbm, k_ref), (v_hbm, v_ref),
            (x_hbm, x_ref), (wq_hbm, wq_ref), (wo_hbm, wo_ref),
        ]):
            cp = pltpu.make_async_copy(src, dst, load_sems.at[i])
            cp.start()
            loads.append(cp)
        k_load, v_load, x_load, wq_load, wo_load = loads

        k_load.wait()
        v_load.wait()
        keff[:, W:W + S, :] = k_ref[...].astype(jnp.bfloat16)
        veff[:, W:W + S, :] = v_ref[...].astype(jnp.bfloat16)

        pl.semaphore_wait(barrier, 2)

        flows = [] if _DEBUG_NO_RDMA else [
            (buf, b, src_lo, dst_lo, tgt)
            for b in range(B)
            for (buf, src_lo, dst_lo, tgt) in [
                (keff, S, 0, right),
                (veff, S, 0, right),
                (keff, W, S + W, left),
                (veff, W, S + W, left),
            ]
        ]
        rdmas = []
        for idx, (buf, b, src_lo, dst_lo, tgt) in enumerate(flows):
            rdma = pltpu.make_async_remote_copy(
                src_ref=buf.at[b, pl.ds(src_lo, W), :],
                dst_ref=buf.at[b, pl.ds(dst_lo, W), :],
                send_sem=send_sems.at[idx],
                recv_sem=recv_sems.at[idx],
                device_id=(tgt,),
                device_id_type=pl.DeviceIdType.MESH,
            )
            rdma.start()
            rdmas.append(rdma)
        left_halo = None if _DEBUG_NO_RDMA else [
            (rdmas[b * 4 + 0], rdmas[b * 4 + 1]) for b in range(B)]
        right_halo = None if _DEBUG_NO_RDMA else [
            (rdmas[b * 4 + 2], rdmas[b * 4 + 3]) for b in range(B)]

        x_load.wait()
        wq_load.wait()
        wo_load.wait()
        wq_b = wq_ref[...].astype(jnp.bfloat16)
        wo_b = wo_ref[...].astype(jnp.bfloat16)
        q_all = []
        for b in range(B):
            xb = x_ref[b].astype(jnp.bfloat16)
            qb = lax.dot_general(
                xb, wq_b, (((1,), (0,)), ((), ())),
                preferred_element_type=jnp.float32,
            )
            q_all.append(qb.astype(jnp.bfloat16))

        BLOCKS = {
            "left": (0, W, 3 * W),
            "mid": (W, S - 2 * W, S),
            "right": (S - W, W, 3 * W),
        }
        biases = {}
        for name, (r0, rw, kw) in BLOCKS.items():
            r_io = lax.broadcasted_iota(jnp.int32, (rw, kw), 0)
            c_io = lax.broadcasted_iota(jnp.int32, (rw, kw), 1)
            band = (c_io - r_io >= 0) & (c_io - r_io <= 2 * W)
            kpos = my * S - W + r0 + c_io
            valid = band & (kpos >= 0) & (kpos < S_GLOBAL)
            biases[name] = jnp.where(valid, 0.0, -1e9).astype(jnp.float32)

        def block(b, name):
            r0, rw, kw = BLOCKS[name]
            kslab = keff[b, pl.ds(r0, kw), :]
            vslab = veff[b, pl.ds(r0, kw), :]
            if _DEBUG_SKIP == "attn":
                ctx = q_all[b][r0:r0 + rw, :]
                out_ref[b, pl.ds(r0, rw), :] = lax.dot_general(
                    ctx, wo_b, (((1,), (0,)), ((), ())),
                    preferred_element_type=jnp.float32,
                ).astype(jnp.bfloat16)
                return
            ctx_heads = []
            for h in range(HQ):
                qh = q_all[b][r0:r0 + rw, h * DH:(h + 1) * DH]
                kh = kslab[:, h * DH:(h + 1) * DH]
                vh = vslab[:, h * DH:(h + 1) * DH]
                s = lax.dot_general(
                    qh, kh, (((1,), (1,)), ((), ())),
                    preferred_element_type=jnp.float32,
                )
                if _DEBUG_SKIP == "softmax":
                    e = s.astype(jnp.bfloat16)
                    inv = 1.0
                else:
                    e = jnp.exp(s * 0.125 + biases[name])
                    inv = 1.0 / lax.dot_general(
                        e.astype(jnp.bfloat16),
                        jnp.ones((kw, 1), jnp.bfloat16),
                        (((1,), (0,)), ((), ())),
                        preferred_element_type=jnp.float32,
                    )
                    e = e.astype(jnp.bfloat16)
                ctx_u = lax.dot_general(
                    e, vh, (((1,), (0,)), ((), ())),
                    preferred_element_type=jnp.float32,
                )
                ctx_heads.append((ctx_u * inv).astype(jnp.bfloat16))
            ctx = jnp.concatenate(ctx_heads, axis=1)
            out_ref[b, pl.ds(r0, rw), :] = lax.dot_general(
                ctx, wo_b, (((1,), (0,)), ((), ())),
                preferred_element_type=jnp.float32,
            ).astype(jnp.bfloat16)

        for b in range(B):
            block(b, "mid")
        for b in range(B):
            if left_halo is not None:
                for r in left_halo[b]:
                    r.wait_recv()
            block(b, "left")
        for b in range(B):
            if right_halo is not None:
                for r in right_halo[b]:
                    r.wait_recv()
            block(b, "right")

        for r in rdmas:
            r.wait_send()

    return pl.pallas_call(
        body,
        out_shape=jax.ShapeDtypeStruct((B, S, E), jnp.bfloat16),
        in_specs=[pl.BlockSpec(memory_space=pl.ANY)] * 5,
        out_specs=pl.BlockSpec(memory_space=pltpu.VMEM),
        scratch_shapes=[
            pltpu.VMEM((B, S, E), jnp.float32),
            pltpu.VMEM((E, HD), jnp.float32),
            pltpu.VMEM((B, S, HD), jnp.float32),
            pltpu.VMEM((B, S, HD), jnp.float32),
            pltpu.VMEM((HD, E), jnp.float32),
            pltpu.VMEM((B, S_EXT, HD), jnp.bfloat16),
            pltpu.VMEM((B, S_EXT, HD), jnp.bfloat16),
            pltpu.SemaphoreType.DMA((5,)),
            pltpu.SemaphoreType.DMA((4 * B,)),
            pltpu.SemaphoreType.DMA((4 * B,)),
        ],
        compiler_params=pltpu.CompilerParams(collective_id=0),
    )(x, Wq, K2, V2, Wo)
